# baseline (speedup 1.0000x reference)
import math

import jax
import jax.numpy as jnp
import numpy as np
from jax.experimental import pallas as pl
from jax.experimental.pallas import tpu as pltpu

SEQ_LEN = 96
PRED_LEN = 96
CHANNELS = 7
N_PERIOD = 3
PERIOD_NUM = (4, 2, 1)
TOPM = 20
D_MODEL = 512
N_TRAIN = 16384
BSZ = 64
FEAT = SEQ_LEN * CHANNELS
WIN = SEQ_LEN + PRED_LEN - 1

TILE_N = 1024


def _build_pool_mats():
    mats = []
    for g in PERIOD_NUM:
        P = np.zeros((SEQ_LEN, SEQ_LEN), np.float32)
        for t in range(SEQ_LEN):
            w = t // g
            P[t, w * g:(w + 1) * g] = 1.0 / g
        M = P - P[SEQ_LEN - 1:SEQ_LEN, :]
        mats.append(M)
    return np.stack(mats)


_M_NP = _build_pool_mats()
_KRONT_NP = np.stack([np.kron(m, np.eye(CHANNELS, dtype=np.float32)).T
                      for m in _M_NP])


def _sim_kernel(x_ref, k_ref, w_ref, b_ref, idx_ref, tr_ref, sim_ref, bxn_ref):
    i = pl.program_id(0)

    @pl.when(i == 0)
    def _():
        xb = x_ref[...]
        for p in range(N_PERIOD):
            xmg = jax.lax.dot(xb, k_ref[p], preferred_element_type=jnp.float32,
                              precision=jax.lax.Precision.HIGHEST)
            bx = jax.lax.dot(xmg, w_ref[...], preferred_element_type=jnp.float32)
            bx = bx + b_ref[...]
            nrm = jnp.sqrt(jnp.sum(bx * bx, axis=1, keepdims=True))
            bxn_ref[p] = bx / jnp.maximum(nrm, 1e-12)

    tile = tr_ref[...]
    jcol = i * TILE_N + jax.lax.broadcasted_iota(jnp.int32, (BSZ, TILE_N), 1)
    idx = idx_ref[...]
    masked = (jcol >= idx - WIN) & (jcol <= idx + WIN)
    for p in range(N_PERIOD):
        mg = jax.lax.dot(tile, k_ref[p], preferred_element_type=jnp.float32,
                         precision=jax.lax.Precision.HIGHEST)
        ax = jax.lax.dot(mg, w_ref[...], preferred_element_type=jnp.float32)
        ax = ax + b_ref[...]
        nrm = jnp.sqrt(jnp.sum(ax * ax, axis=1, keepdims=True))
        axn = ax / jnp.maximum(nrm, 1e-12)
        s = jax.lax.dot_general(bxn_ref[p], axn, (((1,), (1,)), ((), ())),
                                preferred_element_type=jnp.float32)
        sim_ref[p] = jnp.where(masked, -jnp.inf, s)


def _topk_kernel(sim_ref, idx_ref):
    v = sim_ref[...]
    rows = v.shape[0]
    iota = jax.lax.broadcasted_iota(jnp.int32, v.shape, 1)
    col20 = jax.lax.broadcasted_iota(jnp.int32, (rows, TOPM), 1)

    def body(t, carry):
        vv, acc = carry
        mx = jnp.max(vv, axis=1, keepdims=True)
        cand = jnp.where(vv == mx, iota, jnp.int32(2 ** 30))
        amin = jnp.min(cand, axis=1, keepdims=True)
        acc = jnp.where(col20 == t, amin, acc)
        vv = jnp.where(iota == amin, -jnp.inf, vv)
        return vv, acc

    _, acc = jax.lax.fori_loop(
        0, TOPM, body, (v, jnp.zeros((rows, TOPM), jnp.int32)))
    idx_ref[...] = acc


def _gather_kernel(pidx_ref, tr_ref, y_ref, k_ref, v_ref):
    del pidx_ref
    k_ref[...] = tr_ref[...]
    v_ref[...] = y_ref[...]


def _attn_kernel(m_ref, x_ref, k_ref, v_ref, wq_ref, bq_ref, wk_ref, bk_ref,
                 e_ref, out_ref):
    mp = m_ref[0]
    xb = x_ref[0]
    kr = k_ref[0]
    vr = v_ref[0]

    q_mg = jax.lax.dot(mp, xb, preferred_element_type=jnp.float32, precision=jax.lax.Precision.HIGHEST)
    q_emb = jax.lax.dot_general(q_mg, wq_ref[...], (((1,), (1,)), ((), ())),
                                preferred_element_type=jnp.float32, precision=jax.lax.Precision.HIGHEST)
    q_emb = q_emb + bq_ref[...]

    kmg = jax.lax.dot_general(mp, kr, (((1,), (1,)), ((), ())),
                              preferred_element_type=jnp.float32, precision=jax.lax.Precision.HIGHEST)
    k_emb = jax.lax.dot_general(kmg, wk_ref[...], (((2,), (1,)), ((), ())),
                                preferred_element_type=jnp.float32, precision=jax.lax.Precision.HIGHEST)
    k_emb = k_emb + bk_ref[...]
    k_flat = k_emb.reshape(SEQ_LEN * TOPM, D_MODEL)

    attn = jax.lax.dot_general(q_emb, k_flat, (((1,), (1,)), ((), ())),
                               preferred_element_type=jnp.float32, precision=jax.lax.Precision.HIGHEST)
    attn = attn * (1.0 / math.sqrt(D_MODEL))
    mx = jnp.max(attn, axis=1, keepdims=True)
    ea = jnp.exp(attn - mx)
    attn = ea / jnp.sum(ea, axis=1, keepdims=True)

    colsum = jnp.sum(attn, axis=0, keepdims=True)
    nw = jax.lax.dot_general(e_ref[...], colsum, (((1,), (1,)), ((), ())),
                             preferred_element_type=jnp.float32, precision=jax.lax.Precision.HIGHEST)
    nmx = jnp.max(nw, axis=0, keepdims=True)
    en = jnp.exp(nw - nmx)
    nw = en / jnp.sum(en, axis=0, keepdims=True)

    wsum = jnp.sum(vr * nw[:, :, None], axis=0)
    out_ref[0, 0] = jax.lax.dot(mp, wsum, preferred_element_type=jnp.float32, precision=jax.lax.Precision.HIGHEST)


def _run_sim(x_flat, kronT, W_simT, b_row, idx_col, train_flat):
    n_tiles = N_TRAIN // TILE_N
    return pl.pallas_call(
        _sim_kernel,
        grid=(n_tiles,),
        in_specs=[
            pl.BlockSpec((BSZ, FEAT), lambda i: (0, 0)),
            pl.BlockSpec((N_PERIOD, FEAT, FEAT), lambda i: (0, 0, 0)),
            pl.BlockSpec((FEAT, FEAT), lambda i: (0, 0)),
            pl.BlockSpec((1, FEAT), lambda i: (0, 0)),
            pl.BlockSpec((BSZ, 1), lambda i: (0, 0)),
            pl.BlockSpec((TILE_N, FEAT), lambda i: (i, 0)),
        ],
        out_specs=pl.BlockSpec((N_PERIOD, BSZ, TILE_N), lambda i: (0, 0, i)),
        out_shape=jax.ShapeDtypeStruct((N_PERIOD, BSZ, N_TRAIN), jnp.float32),
        scratch_shapes=[pltpu.VMEM((N_PERIOD, BSZ, FEAT), jnp.float32)],
    )(x_flat, kronT, W_simT, b_row, idx_col, train_flat)


def _run_topk(sim2):
    row_blk = 32
    return pl.pallas_call(
        _topk_kernel,
        grid=(N_PERIOD * BSZ // row_blk,),
        in_specs=[pl.BlockSpec((row_blk, N_TRAIN), lambda i: (i, 0))],
        out_specs=pl.BlockSpec((row_blk, TOPM), lambda i: (i, 0)),
        out_shape=jax.ShapeDtypeStruct((N_PERIOD * BSZ, TOPM), jnp.int32),
    )(sim2)


def _run_gather(idx_flat, train_data_all, y_data_all):
    n_gather = N_PERIOD * BSZ * TOPM
    return pl.pallas_call(
        _gather_kernel,
        grid_spec=pltpu.PrefetchScalarGridSpec(
            num_scalar_prefetch=1,
            grid=(n_gather,),
            in_specs=[
                pl.BlockSpec((1, SEQ_LEN, CHANNELS), lambda i, pidx: (pidx[i], 0, 0)),
                pl.BlockSpec((1, PRED_LEN, CHANNELS), lambda i, pidx: (pidx[i], 0, 0)),
            ],
            out_specs=[
                pl.BlockSpec((1, SEQ_LEN, CHANNELS), lambda i, pidx: (i, 0, 0)),
                pl.BlockSpec((1, PRED_LEN, CHANNELS), lambda i, pidx: (i, 0, 0)),
            ],
        ),
        out_shape=[
            jax.ShapeDtypeStruct((n_gather, SEQ_LEN, CHANNELS), jnp.float32),
            jax.ShapeDtypeStruct((n_gather, PRED_LEN, CHANNELS), jnp.float32),
        ],
    )(idx_flat, train_data_all, y_data_all)


def _run_attn(M, x, kr4, vr4, Wq, bq, Wk, bk):
    e_sel = (jnp.arange(SEQ_LEN * TOPM, dtype=jnp.int32)[None, :] % TOPM
             == jnp.arange(TOPM, dtype=jnp.int32)[:, None]).astype(jnp.float32)

    return pl.pallas_call(
        _attn_kernel,
        grid=(N_PERIOD, BSZ),
        in_specs=[
            pl.BlockSpec((1, SEQ_LEN, SEQ_LEN), lambda p, b: (p, 0, 0)),
            pl.BlockSpec((1, SEQ_LEN, CHANNELS), lambda p, b: (b, 0, 0)),
            pl.BlockSpec((1, TOPM, SEQ_LEN, CHANNELS), lambda p, b: (p * BSZ + b, 0, 0, 0)),
            pl.BlockSpec((1, TOPM, PRED_LEN, CHANNELS), lambda p, b: (p * BSZ + b, 0, 0, 0)),
            pl.BlockSpec((D_MODEL, CHANNELS), lambda p, b: (0, 0)),
            pl.BlockSpec((1, D_MODEL), lambda p, b: (0, 0)),
            pl.BlockSpec((D_MODEL, CHANNELS), lambda p, b: (0, 0)),
            pl.BlockSpec((1, D_MODEL), lambda p, b: (0, 0)),
            pl.BlockSpec((TOPM, SEQ_LEN * TOPM), lambda p, b: (0, 0)),
        ],
        out_specs=pl.BlockSpec((1, 1, PRED_LEN, CHANNELS), lambda p, b: (p, b, 0, 0)),
        out_shape=jax.ShapeDtypeStruct((N_PERIOD, BSZ, PRED_LEN, CHANNELS), jnp.float32),
    )(M, x, kr4, vr4, Wq, bq.reshape(1, D_MODEL), Wk, bk.reshape(1, D_MODEL), e_sel)


def kernel(x, index, train_data_all, y_data_all, W_sim, b_sim, Wq, bq, Wk, bk):
    x_flat = x.reshape(BSZ, FEAT)
    train_flat = train_data_all.reshape(N_TRAIN, FEAT)
    M = jnp.asarray(_M_NP)

    idx_col = index.astype(jnp.int32).reshape(BSZ, 1)
    b_row = b_sim.reshape(1, FEAT)

    sim = _run_sim(x_flat, jnp.asarray(_KRONT_NP), W_sim.T, b_row, idx_col,
                   train_flat)
    sim2 = sim.reshape(N_PERIOD * BSZ, N_TRAIN)
    topk = _run_topk(sim2)
    idx_flat = topk.reshape(-1)
    kraw, vraw = _run_gather(idx_flat, train_data_all, y_data_all)
    kr4 = kraw.reshape(N_PERIOD * BSZ, TOPM, SEQ_LEN, CHANNELS)
    vr4 = vraw.reshape(N_PERIOD * BSZ, TOPM, PRED_LEN, CHANNELS)
    return _run_attn(M, x, kr4, vr4, Wq, bq, Wk, bk)
